# Initial kernel scaffold; baseline (speedup 1.0000x reference)
import jax
import jax.numpy as jnp
from jax.experimental import pallas as pl


def kernel(preds, truths):
    raise NotImplementedError("write your pallas kernel here")


# speedup vs baseline: 13.3845x; 13.3845x over previous
import functools

import jax
import jax.numpy as jnp
from jax import lax
from jax.experimental import pallas as pl
from jax.experimental.pallas import tpu as pltpu
from jax.experimental.pallas import tpu_sc as plsc

NCLS = 150
STRIDE = 160
NBINS = NCLS * STRIDE
NW = 32
NPIX = 8 * 512 * 512
PER_W = NPIX // NW
CHUNK = 8192
NCHUNK = PER_W // CHUNK
LANES = 16


def _hist_body(preds_hbm, truths_hbm, out_hbm, pbuf, tbuf, hist):
    cid = lax.axis_index("c")
    sid = lax.axis_index("s")
    wid = sid * 2 + cid
    base = wid * PER_W

    zeros = jnp.zeros((LANES,), jnp.int32)

    def zero_body(i, _):
        hist[pl.ds(i * LANES, LANES)] = zeros
        return 0

    lax.fori_loop(0, NBINS // LANES, zero_body, 0)

    def chunk_body(c, _):
        off = pl.multiple_of(base + c * CHUNK, CHUNK)
        pltpu.sync_copy(preds_hbm.at[pl.ds(off, CHUNK)], pbuf)
        pltpu.sync_copy(truths_hbm.at[pl.ds(off, CHUNK)], tbuf)

        def vec_body(i, _):
            p = pbuf[pl.ds(i * LANES, LANES)]
            t = tbuf[pl.ds(i * LANES, LANES)]
            idx = t * STRIDE + p
            cnt, last = plsc.scan_count(idx)
            plsc.addupdate_scatter(hist, [idx], cnt, mask=last)
            return 0

        lax.fori_loop(0, CHUNK // LANES, vec_body, 0)
        return 0

    lax.fori_loop(0, NCHUNK, chunk_body, 0)
    pltpu.sync_copy(hist, out_hbm.at[wid])


@functools.partial(jax.jit, static_argnames=())
def _sc_hist(preds_flat, truths_flat):
    mesh = plsc.VectorSubcoreMesh(core_axis_name="c", subcore_axis_name="s")
    return pl.kernel(
        _hist_body,
        out_type=jax.ShapeDtypeStruct((NW, NBINS), jnp.int32),
        mesh=mesh,
        compiler_params=pltpu.CompilerParams(needs_layout_passes=False),
        scratch_types=[
            pltpu.VMEM((CHUNK,), jnp.int32),
            pltpu.VMEM((CHUNK,), jnp.int32),
            pltpu.VMEM((NBINS,), jnp.int32),
        ],
    )(preds_flat, truths_flat)


def _metrics_body(part_ref, out_ref):
    cm = jnp.sum(part_ref[...].astype(jnp.float32), axis=0)
    row_i = lax.broadcasted_iota(jnp.int32, (NCLS, STRIDE), 0)
    col_i = lax.broadcasted_iota(jnp.int32, (NCLS, STRIDE), 1)
    eye = row_i == col_i
    d = jnp.sum(jnp.where(eye, cm, 0.0), axis=1)
    rs = jnp.sum(cm, axis=1)
    cs = jnp.sum(cm, axis=0)[:NCLS]
    exist = cs > 0.0
    nex = jnp.sum(exist.astype(jnp.float32))
    safe_cs = jnp.where(exist, cs, 1.0)
    caa = jnp.sum(jnp.where(exist, d / safe_cs, 0.0)) / nex
    denom = cs + rs - d
    safe_den = jnp.where(exist, denom, 1.0)
    miou = jnp.sum(jnp.where(exist, d / safe_den, 0.0)) / nex
    tot = jnp.sum(d) / jnp.sum(rs)
    lane = lax.broadcasted_iota(jnp.int32, (1, 128), 1)
    vec = (
        jnp.where(lane == 0, miou, 0.0)
        + jnp.where(lane == 1, tot, 0.0)
        + jnp.where(lane == 2, caa, 0.0)
    )
    out_ref[...] = vec


def kernel(preds, truths):
    preds_flat = preds.reshape(-1)
    truths_flat = truths.reshape(-1)
    part = _sc_hist(preds_flat, truths_flat)
    part3 = part.reshape(NW, NCLS, STRIDE)
    out = pl.pallas_call(
        _metrics_body,
        out_shape=jax.ShapeDtypeStruct((1, 128), jnp.float32),
    )(part3)
    return out[0, :3]


# speedup vs baseline: 17.9263x; 1.3393x over previous
import functools

import jax
import jax.numpy as jnp
from jax import lax
from jax.experimental import pallas as pl
from jax.experimental.pallas import tpu as pltpu
from jax.experimental.pallas import tpu_sc as plsc

NCLS = 150
STRIDE = 160
NBINS = NCLS * STRIDE
NW = 32
NPIX = 8 * 512 * 512
PER_W = NPIX // NW
CHUNK = 8192
NCHUNK = PER_W // CHUNK
LANES = 16


def _hist_body(preds_hbm, truths_hbm, out_hbm, pbuf, tbuf, hist):
    cid = lax.axis_index("c")
    sid = lax.axis_index("s")
    wid = sid * 2 + cid
    base = wid * PER_W

    zeros = jnp.zeros((LANES,), jnp.int32)
    ones = jnp.ones((LANES,), jnp.int32)

    def zero_body(i, _):
        hist[pl.ds(i * LANES, LANES)] = zeros
        return 0

    lax.fori_loop(0, NBINS // LANES, zero_body, 0)

    def chunk_body(c, _):
        off = pl.multiple_of(base + c * CHUNK, CHUNK)
        pltpu.sync_copy(preds_hbm.at[pl.ds(off, CHUNK)], pbuf)
        pltpu.sync_copy(truths_hbm.at[pl.ds(off, CHUNK)], tbuf)

        def vec_body(i, _):
            p = pbuf[pl.ds(i * LANES, LANES)]
            t = tbuf[pl.ds(i * LANES, LANES)]
            idx = t * STRIDE + p
            plsc.addupdate_scatter(hist, [idx], ones)
            return 0

        lax.fori_loop(0, CHUNK // LANES, vec_body, 0)
        return 0

    lax.fori_loop(0, NCHUNK, chunk_body, 0)
    pltpu.sync_copy(hist, out_hbm.at[wid])


@functools.partial(jax.jit, static_argnames=())
def _sc_hist(preds_flat, truths_flat):
    mesh = plsc.VectorSubcoreMesh(core_axis_name="c", subcore_axis_name="s")
    return pl.kernel(
        _hist_body,
        out_type=jax.ShapeDtypeStruct((NW, NBINS), jnp.int32),
        mesh=mesh,
        compiler_params=pltpu.CompilerParams(needs_layout_passes=False),
        scratch_types=[
            pltpu.VMEM((CHUNK,), jnp.int32),
            pltpu.VMEM((CHUNK,), jnp.int32),
            pltpu.VMEM((NBINS,), jnp.int32),
        ],
    )(preds_flat, truths_flat)


def _metrics_body(part_ref, out_ref):
    cm = jnp.sum(part_ref[...].astype(jnp.float32), axis=0)
    row_i = lax.broadcasted_iota(jnp.int32, (NCLS, STRIDE), 0)
    col_i = lax.broadcasted_iota(jnp.int32, (NCLS, STRIDE), 1)
    eye = row_i == col_i
    d = jnp.sum(jnp.where(eye, cm, 0.0), axis=1)
    rs = jnp.sum(cm, axis=1)
    cs = jnp.sum(cm, axis=0)[:NCLS]
    exist = cs > 0.0
    nex = jnp.sum(exist.astype(jnp.float32))
    safe_cs = jnp.where(exist, cs, 1.0)
    caa = jnp.sum(jnp.where(exist, d / safe_cs, 0.0)) / nex
    denom = cs + rs - d
    safe_den = jnp.where(exist, denom, 1.0)
    miou = jnp.sum(jnp.where(exist, d / safe_den, 0.0)) / nex
    tot = jnp.sum(d) / jnp.sum(rs)
    lane = lax.broadcasted_iota(jnp.int32, (1, 128), 1)
    vec = (
        jnp.where(lane == 0, miou, 0.0)
        + jnp.where(lane == 1, tot, 0.0)
        + jnp.where(lane == 2, caa, 0.0)
    )
    out_ref[...] = vec


def kernel(preds, truths):
    preds_flat = preds.reshape(-1)
    truths_flat = truths.reshape(-1)
    part = _sc_hist(preds_flat, truths_flat)
    part3 = part.reshape(NW, NCLS, STRIDE)
    out = pl.pallas_call(
        _metrics_body,
        out_shape=jax.ShapeDtypeStruct((1, 128), jnp.float32),
    )(part3)
    return out[0, :3]


# speedup vs baseline: 19.2050x; 1.0713x over previous
import functools

import jax
import jax.numpy as jnp
from jax import lax
from jax.experimental import pallas as pl
from jax.experimental.pallas import tpu as pltpu
from jax.experimental.pallas import tpu_sc as plsc

NCLS = 150
STRIDE = 160
NBINS = NCLS * STRIDE
NW = 32
NPIX = 8 * 512 * 512
PER_W = NPIX // NW
CHUNK = 8192
NCHUNK = PER_W // CHUNK
LANES = 16


def _hist_body(preds_hbm, truths_hbm, out_hbm, pbuf, tbuf, hist):
    cid = lax.axis_index("c")
    sid = lax.axis_index("s")
    wid = sid * 2 + cid
    base = wid * PER_W

    zeros = jnp.zeros((LANES,), jnp.int32)
    ones = jnp.ones((LANES,), jnp.int32)

    ZUNROLL = 10

    def zero_body(i, _):
        for u in range(ZUNROLL):
            hist[pl.ds((i * ZUNROLL + u) * LANES, LANES)] = zeros
        return 0

    lax.fori_loop(0, NBINS // LANES // ZUNROLL, zero_body, 0)

    UNROLL = 8

    def chunk_body(c, _):
        off = pl.multiple_of(base + c * CHUNK, CHUNK)
        pltpu.sync_copy(preds_hbm.at[pl.ds(off, CHUNK)], pbuf)
        pltpu.sync_copy(truths_hbm.at[pl.ds(off, CHUNK)], tbuf)

        def vec_body(i, _):
            for u in range(UNROLL):
                b = (i * UNROLL + u) * LANES
                p = pbuf[pl.ds(b, LANES)]
                t = tbuf[pl.ds(b, LANES)]
                idx = t * STRIDE + p
                plsc.addupdate_scatter(hist, [idx], ones)
            return 0

        lax.fori_loop(0, CHUNK // LANES // UNROLL, vec_body, 0)
        return 0

    lax.fori_loop(0, NCHUNK, chunk_body, 0)
    pltpu.sync_copy(hist, out_hbm.at[wid])


@functools.partial(jax.jit, static_argnames=())
def _sc_hist(preds_flat, truths_flat):
    mesh = plsc.VectorSubcoreMesh(core_axis_name="c", subcore_axis_name="s")
    return pl.kernel(
        _hist_body,
        out_type=jax.ShapeDtypeStruct((NW, NBINS), jnp.int32),
        mesh=mesh,
        compiler_params=pltpu.CompilerParams(needs_layout_passes=False),
        scratch_types=[
            pltpu.VMEM((CHUNK,), jnp.int32),
            pltpu.VMEM((CHUNK,), jnp.int32),
            pltpu.VMEM((NBINS,), jnp.int32),
        ],
    )(preds_flat, truths_flat)


def _metrics_body(part_ref, out_ref):
    cm = jnp.sum(part_ref[...].astype(jnp.float32), axis=0)
    row_i = lax.broadcasted_iota(jnp.int32, (NCLS, STRIDE), 0)
    col_i = lax.broadcasted_iota(jnp.int32, (NCLS, STRIDE), 1)
    eye = row_i == col_i
    d = jnp.sum(jnp.where(eye, cm, 0.0), axis=1)
    rs = jnp.sum(cm, axis=1)
    cs = jnp.sum(cm, axis=0)[:NCLS]
    exist = cs > 0.0
    nex = jnp.sum(exist.astype(jnp.float32))
    safe_cs = jnp.where(exist, cs, 1.0)
    caa = jnp.sum(jnp.where(exist, d / safe_cs, 0.0)) / nex
    denom = cs + rs - d
    safe_den = jnp.where(exist, denom, 1.0)
    miou = jnp.sum(jnp.where(exist, d / safe_den, 0.0)) / nex
    tot = jnp.sum(d) / jnp.sum(rs)
    lane = lax.broadcasted_iota(jnp.int32, (1, 128), 1)
    vec = (
        jnp.where(lane == 0, miou, 0.0)
        + jnp.where(lane == 1, tot, 0.0)
        + jnp.where(lane == 2, caa, 0.0)
    )
    out_ref[...] = vec


def kernel(preds, truths):
    preds_flat = preds.reshape(-1)
    truths_flat = truths.reshape(-1)
    part = _sc_hist(preds_flat, truths_flat)
    part3 = part.reshape(NW, NCLS, STRIDE)
    out = pl.pallas_call(
        _metrics_body,
        out_shape=jax.ShapeDtypeStruct((1, 128), jnp.float32),
    )(part3)
    return out[0, :3]
